# baseline (speedup 1.0000x reference)
import functools

import jax
import jax.numpy as jnp
from jax import lax
from jax.experimental import pallas as pl
from jax.experimental.pallas import tpu as pltpu
from jax.experimental.pallas import tpu_sc as plsc

L = 16
D = 128
DV = D // L
NSEG = 1024
NC = 2
NS = 16
NW = NC * NS
SEG_PER_W = NSEG // NW
CHUNK = 128


def _sc_kernel(feats, starts, wvec, bvec):
    n_rows = feats.shape[0]
    nmax = n_rows - CHUNK

    mesh = plsc.VectorSubcoreMesh(core_axis_name="c", subcore_axis_name="s")

    @functools.partial(
        pl.kernel,
        mesh=mesh,
        out_type=jax.ShapeDtypeStruct((NSEG, 2 * D), jnp.float32),
        scratch_types=[
            pltpu.VMEM((starts.shape[0],), jnp.int32),
            pltpu.VMEM((D,), jnp.float32),
            pltpu.VMEM((L,), jnp.float32),
            pltpu.VMEM((CHUNK, D), jnp.float32),
            pltpu.VMEM((CHUNK, D), jnp.float32),
            pltpu.VMEM((CHUNK, D), jnp.float32),
            pltpu.VMEM((L * L,), jnp.float32),
            pltpu.VMEM((CHUNK,), jnp.float32),
            pltpu.VMEM((SEG_PER_W, 2 * D), jnp.float32),
            pltpu.SemaphoreType.DMA,
            pltpu.SemaphoreType.DMA,
        ],
        compiler_params=pltpu.CompilerParams(needs_layout_passes=False),
    )
    def body(feats_hbm, starts_hbm, w_hbm, b_hbm, out_hbm,
             starts_v, w_v, b_v, buf_a, buf_b, buf_o, stage, gates,
             out_stage, sem_a, sem_b):
        wid = lax.axis_index("s") * NC + lax.axis_index("c")
        pltpu.sync_copy(starts_hbm, starts_v)
        pltpu.sync_copy(w_hbm, w_v)
        pltpu.sync_copy(b_hbm, b_v)

        wreg = [w_v[pl.ds(t * L, L)] for t in range(DV)]
        breg = b_v[...]
        lane = lax.iota(jnp.int32, L)
        gidx = lane * L + (L - 1)

        seg0 = wid * SEG_PER_W

        def seg_range(sl):
            sv = starts_v[pl.ds(seg0 + sl, L)]
            return sv[0], sv[1]

        def dma_start(st):
            return pl.multiple_of(jnp.minimum(st & (-8), nmax), 8)

        def issue(sl, buf, sem):
            st, _ = seg_range(sl)
            pltpu.async_copy(
                feats_hbm.at[pl.ds(dma_start(st), CHUNK)], buf, sem)

        def wait(sl, buf, sem):
            st, _ = seg_range(sl)
            pltpu.make_async_copy(
                feats_hbm.at[pl.ds(dma_start(st), CHUNK)], buf, sem).wait()

        def do_chunk(buf, p_lo, p_hi, acc):
            g0 = p_lo // L
            g1 = (p_hi + (L - 1)) // L

            def p1(g, _):
                base = g * L
                for j in range(L):
                    rv = [buf[base + j, pl.ds(t * L, L)] for t in range(DV)]
                    part = rv[0] * wreg[0]
                    for t in range(1, DV):
                        part = part + rv[t] * wreg[t]
                    stage[pl.ds(j * L, L)] = plsc.cumsum(part)
                dots = plsc.load_gather(stage, [gidx])
                sig = 1.0 / (1.0 + jnp.exp(-(dots + breg)))
                gates[pl.ds(base, L)] = sig
                return 0

            lax.fori_loop(g0, g1, p1, 0)

            def p2(p, c):
                accs = list(c[:DV])
                accm = list(c[DV:])
                gv = plsc.load_gather(gates, [jnp.broadcast_to(p, (L,))])
                rv = [buf[p, pl.ds(t * L, L)] for t in range(DV)]
                for t in range(DV):
                    accs[t] = accs[t] + gv * rv[t]
                    accm[t] = jnp.maximum(accm[t], rv[t])
                return tuple(accs) + tuple(accm)

            return lax.fori_loop(p_lo, p_hi, p2, acc)

        zero = jnp.zeros((L,), jnp.float32)
        ninf = jnp.full((L,), -jnp.inf, jnp.float32)
        acc_init = tuple(zero for _ in range(DV)) + tuple(
            ninf for _ in range(DV))

        def seg_process(sl, buf):
            st, en = seg_range(sl)
            cs = dma_start(st)
            hi0 = jnp.minimum(en, cs + CHUNK)
            acc = do_chunk(buf, st - cs, hi0 - cs, acc_init)

            def ovf(c, a):
                lo_c = st * 0 + cs + c * CHUNK
                cs_c = pl.multiple_of(jnp.minimum(lo_c, nmax), 8)
                pltpu.sync_copy(feats_hbm.at[pl.ds(cs_c, CHUNK)], buf_o)
                hi_c = jnp.minimum(en, lo_c + CHUNK)
                return do_chunk(buf_o, lo_c - cs_c, hi_c - cs_c, a)

            novf = jnp.maximum((en - cs - 1) // CHUNK, 0)
            acc = lax.fori_loop(1, 1 + novf, ovf, acc)

            for t in range(DV):
                out_stage[sl, pl.ds(t * L, L)] = acc[t]
                out_stage[sl, pl.ds(D + t * L, L)] = acc[DV + t]

        issue(0, buf_a, sem_a)

        def pair(it, _):
            sl_a = 2 * it
            wait(sl_a, buf_a, sem_a)
            issue(sl_a + 1, buf_b, sem_b)
            seg_process(sl_a, buf_a)
            wait(sl_a + 1, buf_b, sem_b)

            @pl.when(it < SEG_PER_W // 2 - 1)
            def _():
                issue(sl_a + 2, buf_a, sem_a)

            seg_process(sl_a + 1, buf_b)
            return 0

        lax.fori_loop(0, SEG_PER_W // 2, pair, 0)
        pltpu.sync_copy(out_stage, out_hbm.at[pl.ds(seg0, SEG_PER_W)])

    return body(feats, starts, wvec, bvec)


def kernel(feats, segment_ids, W, b):
    starts = jnp.searchsorted(
        segment_ids, jnp.arange(NSEG + 1, dtype=segment_ids.dtype)
    ).astype(jnp.int32)
    starts = jnp.pad(starts, (0, L - 1))
    wvec = W.reshape(D).astype(jnp.float32)
    bvec = jnp.broadcast_to(b.astype(jnp.float32), (L,))
    return _sc_kernel(feats, starts, wvec, bvec)
